# Initial kernel scaffold; baseline (speedup 1.0000x reference)
import functools

import jax
import jax.numpy as jnp
from jax import lax
from jax.experimental import pallas as pl
from jax.experimental.pallas import tpu as pltpu
from jax.experimental.pallas import tpu_sc as plsc

NC = 2
NS = 16
LANES = 16
CHUNK = 128
ROWS_PER_TILE = 640

_F32 = jnp.float32
_I32 = jnp.int32


def _zero_rows(zb, d):
    z = jnp.zeros((LANES,), _F32)

    def body(i, _):
        for k in range(d // LANES):
            zb[i, pl.ds(k * LANES, LANES)] = z
        return 0

    lax.fori_loop(0, 128, body, 0)


def _make_sc_deg(n_pad, n_chunk_rows):
    seg = n_pad // NS

    @functools.partial(
        pl.kernel,
        mesh=plsc.VectorSubcoreMesh(core_axis_name="c", subcore_axis_name="s"),
        out_type=jax.ShapeDtypeStruct((n_pad,), _F32),
        scratch_types=[
            pltpu.VMEM((n_chunk_rows, CHUNK), _I32),
            pltpu.VMEM((n_chunk_rows, CHUNK), _F32),
            pltpu.VMEM((seg,), _F32),
            pltpu.VMEM_SHARED((n_pad,), _F32),
        ],
    )
    def deg_kernel(dst2d, ew2d, dis_out, dst_all, ew_all, buf, acc):
        c = lax.axis_index("c")
        s = lax.axis_index("s")

        @pl.when(c == 0)
        def _():
            z = jnp.zeros((LANES,), _F32)

            def zb(i, _):
                buf[pl.ds(i * LANES, LANES)] = z
                return 0

            lax.fori_loop(0, seg // LANES, zb, 0)
            pltpu.sync_copy(buf, acc.at[pl.ds(s * seg, seg)])
            plsc.subcore_barrier()

            pltpu.sync_copy(dst2d.at[pl.ds(s * n_chunk_rows, n_chunk_rows)], dst_all)
            pltpu.sync_copy(ew2d.at[pl.ds(s * n_chunk_rows, n_chunk_rows)], ew_all)

            def chunk(g, _):
                pltpu.sync_copy(ew_all.at[g], acc.at[dst_all.at[g]], add=True)
                return 0

            lax.fori_loop(0, n_chunk_rows, chunk, 0)
            plsc.subcore_barrier()

            pltpu.sync_copy(acc.at[pl.ds(s * seg, seg)], buf)

            def newton(i, _):
                sl = pl.ds(i * LANES, LANES)
                d = buf[sl] + 1.0
                pos = d > 0.0
                bits = plsc.bitcast(d, _I32)
                y = plsc.bitcast(
                    jnp.int32(0x5F3759DF) - lax.shift_right_arithmetic(bits, 1), _F32
                )
                half = d * 0.5
                for _it in range(4):
                    y = y * (1.5 - half * y * y)
                buf[sl] = jnp.where(pos, y, 0.0)
                return 0

            lax.fori_loop(0, seg // LANES, newton, 0)
            pltpu.sync_copy(buf, dis_out.at[pl.ds(s * seg, seg)])

    return deg_kernel


def _make_sc_agg(n_pad, d, n_chunk_rows):
    seg = n_pad // NS

    @functools.partial(
        pl.kernel,
        mesh=plsc.VectorSubcoreMesh(core_axis_name="c", subcore_axis_name="s"),
        out_type=jax.ShapeDtypeStruct((NC, n_pad, d), _F32),
        scratch_types=[
            pltpu.VMEM((n_chunk_rows, CHUNK), _I32),
            pltpu.VMEM((n_chunk_rows, CHUNK), _I32),
            pltpu.VMEM((n_chunk_rows, CHUNK), _F32),
            pltpu.VMEM((CHUNK, d), _F32),
            pltpu.VMEM((128, d), _F32),
            pltpu.VMEM_SHARED((n_pad, d), _F32),
            pltpu.SemaphoreType.DMA,
        ],
    )
    def agg_kernel(src2d, dst2d, ew2d, hs, out, src_all, dst_all, ew_all,
                   rows_v, zb, acc, sem):
        c = lax.axis_index("c")
        s = lax.axis_index("s")
        wid = c * NS + s

        _zero_rows(zb, d)
        for i in range(seg // 128):
            pltpu.sync_copy(zb, acc.at[pl.ds(s * seg + i * 128, 128)])
        plsc.subcore_barrier()

        pltpu.sync_copy(src2d.at[pl.ds(wid * n_chunk_rows, n_chunk_rows)], src_all)
        pltpu.sync_copy(dst2d.at[pl.ds(wid * n_chunk_rows, n_chunk_rows)], dst_all)
        pltpu.sync_copy(ew2d.at[pl.ds(wid * n_chunk_rows, n_chunk_rows)], ew_all)

        def chunk(g, _):
            pltpu.async_copy(hs.at[src_all.at[g]], rows_v, sem).wait()

            def scale(j, _):
                cv = jnp.full((LANES,), ew_all[g, j], _F32)
                for k in range(d // LANES):
                    sl = pl.ds(k * LANES, LANES)
                    rows_v[j, sl] = rows_v[j, sl] * cv
                return 0

            lax.fori_loop(0, CHUNK, scale, 0)
            pltpu.sync_copy(rows_v, acc.at[dst_all.at[g]], add=True)
            return 0

        lax.fori_loop(0, n_chunk_rows, chunk, 0)
        plsc.subcore_barrier()

        for i in range(seg // 128):
            r0 = s * seg + i * 128
            pltpu.sync_copy(acc.at[pl.ds(r0, 128)], out.at[c, pl.ds(r0, 128)])

    return agg_kernel


def _tc_scale(x, w, dis_col):
    n = x.shape[0]
    h = w.shape[1]

    def body(x_ref, w_ref, d_ref, o_ref):
        hm = jnp.dot(x_ref[...], w_ref[...], preferred_element_type=_F32)
        o_ref[...] = hm * d_ref[...]

    return pl.pallas_call(
        body, out_shape=jax.ShapeDtypeStruct((n, h), _F32)
    )(x, w, dis_col)


def _tc_mid(p0, p1, hs1, dis_col, b1, w2p):
    n, _h = hs1.shape
    d2 = w2p.shape[1]

    def body(p0_ref, p1_ref, hs_ref, d_ref, b_ref, w_ref, o_ref):
        t = (p0_ref[...] + p1_ref[...] + hs_ref[...]) * d_ref[...] + b_ref[...]
        r = jnp.maximum(t, 0.0)
        o_ref[...] = jnp.dot(r, w_ref[...], preferred_element_type=_F32) * d_ref[...]

    return pl.pallas_call(
        body, out_shape=jax.ShapeDtypeStruct((n, d2), _F32)
    )(p0, p1, hs1, dis_col, b1, w2p)


def _tc_final(q0, q1, hs2p, dis_col, b2):
    n = hs2p.shape[0]
    c_out = b2.shape[0]

    def body(q0_ref, q1_ref, hs_ref, d_ref, b_ref, o_ref):
        t = (q0_ref[...] + q1_ref[...] + hs_ref[...]) * d_ref[...]
        o_ref[...] = t[:, :c_out] + b_ref[...]

    return pl.pallas_call(
        body, out_shape=jax.ShapeDtypeStruct((n, c_out), _F32)
    )(q0, q1, hs2p, dis_col, b2)


def kernel(x, edge_index, edge_attr, W1, b1, W2, b2):
    n, _dim = x.shape
    e = edge_attr.shape[0]
    n_pad = -(-n // (NS * ROWS_PER_TILE)) * (NS * ROWS_PER_TILE)

    e_pad = -(-e // (NC * NS * CHUNK)) * (NC * NS * CHUNK)
    pad = e_pad - e
    src = jnp.pad(edge_index[0], (0, pad)).reshape(-1, CHUNK)
    dst = jnp.pad(edge_index[1], (0, pad)).reshape(-1, CHUNK)
    ew = jnp.pad(edge_attr, (0, pad)).reshape(-1, CHUNK)

    deg_rows = e_pad // (NS * CHUNK)
    agg_rows = e_pad // (NC * NS * CHUNK)

    dis_pad = _make_sc_deg(n_pad, deg_rows)(dst, ew)
    dis_col = dis_pad[:n].reshape(n, 1)

    hs1 = _tc_scale(x, W1, dis_col)
    p = _make_sc_agg(n_pad, W1.shape[1], agg_rows)(src, dst, ew, hs1)

    d2p = -(-W2.shape[1] // LANES) * LANES
    w2p = jnp.pad(W2, ((0, 0), (0, d2p - W2.shape[1])))
    hs2p = _tc_mid(p[0, :n], p[1, :n], hs1, dis_col, b1, w2p)

    q = _make_sc_agg(n_pad, d2p, agg_rows)(src, dst, ew, hs2p)
    return _tc_final(q[0, :n], q[1, :n], hs2p, dis_col, b2)


# speedup vs baseline: 14.4731x; 14.4731x over previous
import functools

import jax
import jax.numpy as jnp
from jax import lax
from jax.experimental import pallas as pl
from jax.experimental.pallas import tpu as pltpu
from jax.experimental.pallas import tpu_sc as plsc

NC = 2
NS = 16
LANES = 16
CHUNK = 128
ROWS_PER_TILE = 640

_F32 = jnp.float32
_I32 = jnp.int32


def _zero_rows(zb, d):
    z = jnp.zeros((LANES,), _F32)

    def body(i, _):
        for k in range(d // LANES):
            zb[i, pl.ds(k * LANES, LANES)] = z
        return 0

    lax.fori_loop(0, 128, body, 0)


def _make_sc_deg(n_pad, n_chunk_rows):
    seg = n_pad // NS

    @functools.partial(
        pl.kernel,
        mesh=plsc.VectorSubcoreMesh(core_axis_name="c", subcore_axis_name="s"),
        out_type=jax.ShapeDtypeStruct((n_pad,), _F32),
        compiler_params=pltpu.CompilerParams(use_tc_tiling_on_sc=False),
        scratch_types=[
            pltpu.VMEM((n_chunk_rows, CHUNK), _I32),
            pltpu.VMEM((n_chunk_rows, CHUNK), _F32),
            pltpu.VMEM((seg,), _F32),
            pltpu.VMEM_SHARED((n_pad,), _F32),
        ],
    )
    def deg_kernel(dst2d, ew2d, dis_out, dst_all, ew_all, buf, acc):
        c = lax.axis_index("c")
        s = lax.axis_index("s")

        @pl.when(c == 0)
        def _():
            z = jnp.zeros((LANES,), _F32)

            def zb(i, _):
                buf[pl.ds(i * LANES, LANES)] = z
                return 0

            lax.fori_loop(0, seg // LANES, zb, 0)
            pltpu.sync_copy(buf, acc.at[pl.ds(s * seg, seg)])
            plsc.subcore_barrier()

            pltpu.sync_copy(dst2d.at[pl.ds(s * n_chunk_rows, n_chunk_rows)], dst_all)
            pltpu.sync_copy(ew2d.at[pl.ds(s * n_chunk_rows, n_chunk_rows)], ew_all)

            def chunk(g, _):
                pltpu.sync_copy(ew_all.at[g], acc.at[dst_all.at[g]], add=True)
                return 0

            lax.fori_loop(0, n_chunk_rows, chunk, 0)
            plsc.subcore_barrier()

            pltpu.sync_copy(acc.at[pl.ds(s * seg, seg)], buf)

            def newton(i, _):
                sl = pl.ds(i * LANES, LANES)
                d = buf[sl] + 1.0
                pos = d > 0.0
                bits = lax.bitcast_convert_type(d, _I32)
                y = lax.bitcast_convert_type(
                    jnp.int32(0x5F3759DF) - lax.shift_right_arithmetic(bits, 1), _F32
                )
                half = d * 0.5
                for _it in range(4):
                    y = y * (1.5 - half * y * y)
                buf[sl] = jnp.where(pos, y, 0.0)
                return 0

            lax.fori_loop(0, seg // LANES, newton, 0)
            pltpu.sync_copy(buf, dis_out.at[pl.ds(s * seg, seg)])

    return deg_kernel


def _make_sc_agg(n_pad, d, n_chunk_rows):
    seg = n_pad // NS

    @functools.partial(
        pl.kernel,
        mesh=plsc.VectorSubcoreMesh(core_axis_name="c", subcore_axis_name="s"),
        out_type=pltpu.HBM((NC, n_pad, d), _F32),
        compiler_params=pltpu.CompilerParams(use_tc_tiling_on_sc=False),
        scratch_types=[
            pltpu.VMEM((n_chunk_rows, CHUNK), _I32),
            pltpu.VMEM((n_chunk_rows, CHUNK), _I32),
            pltpu.VMEM((n_chunk_rows, CHUNK), _F32),
            pltpu.VMEM((CHUNK, d), _F32),
            pltpu.VMEM((128, d), _F32),
            pltpu.VMEM_SHARED((n_pad, d), _F32),
            pltpu.SemaphoreType.DMA,
        ],
    )
    def agg_kernel(src2d, dst2d, ew2d, hs, out, src_all, dst_all, ew_all,
                   rows_v, zb, acc, sem):
        c = lax.axis_index("c")
        s = lax.axis_index("s")
        wid = c * NS + s

        _zero_rows(zb, d)
        for i in range(seg // 128):
            pltpu.sync_copy(zb, acc.at[pl.ds(s * seg + i * 128, 128)])
        plsc.subcore_barrier()

        pltpu.sync_copy(src2d.at[pl.ds(wid * n_chunk_rows, n_chunk_rows)], src_all)
        pltpu.sync_copy(dst2d.at[pl.ds(wid * n_chunk_rows, n_chunk_rows)], dst_all)
        pltpu.sync_copy(ew2d.at[pl.ds(wid * n_chunk_rows, n_chunk_rows)], ew_all)

        def chunk(g, _):
            pltpu.async_copy(hs.at[src_all.at[g]], rows_v, sem).wait()

            def scale(jb, _):
                ewv = ew_all[g, pl.ds(jb * LANES, LANES)]
                for jj in range(LANES):
                    cv = jnp.full((LANES,), ewv[jj], _F32)
                    j = jb * LANES + jj
                    for k in range(d // LANES):
                        sl = pl.ds(k * LANES, LANES)
                        rows_v[j, sl] = rows_v[j, sl] * cv
                return 0

            lax.fori_loop(0, CHUNK // LANES, scale, 0)
            pltpu.sync_copy(rows_v, acc.at[dst_all.at[g]], add=True)
            return 0

        lax.fori_loop(0, n_chunk_rows, chunk, 0)
        plsc.subcore_barrier()

        for i in range(seg // 128):
            r0 = s * seg + i * 128
            pltpu.sync_copy(acc.at[pl.ds(r0, 128)], out.at[c, pl.ds(r0, 128)])

    return agg_kernel


def _tc_scale(x, w, dis_col, n_pad):
    n = x.shape[0]
    h = w.shape[1]

    def body(x_ref, w_ref, d_ref, o_ref):
        hm = jnp.dot(x_ref[...], w_ref[...], preferred_element_type=_F32)
        o_ref[pl.ds(0, n), :] = hm * d_ref[...]
        o_ref[pl.ds(n, n_pad - n), :] = jnp.zeros((n_pad - n, h), _F32)

    return pl.pallas_call(
        body, out_shape=jax.ShapeDtypeStruct((n_pad, h), _F32)
    )(x, w, dis_col)


def _tc_mid(p0, p1, hs1, dis_col, b1, w2p, n_pad):
    n, _h = hs1.shape
    d2 = w2p.shape[1]

    def body(p0_ref, p1_ref, hs_ref, d_ref, b_ref, w_ref, o_ref):
        t = (p0_ref[...] + p1_ref[...] + hs_ref[...]) * d_ref[...] + b_ref[...]
        r = jnp.maximum(t, 0.0)
        o_ref[pl.ds(0, n), :] = (
            jnp.dot(r, w_ref[...], preferred_element_type=_F32) * d_ref[...]
        )
        o_ref[pl.ds(n, n_pad - n), :] = jnp.zeros((n_pad - n, d2), _F32)

    return pl.pallas_call(
        body, out_shape=jax.ShapeDtypeStruct((n_pad, d2), _F32)
    )(p0, p1, hs1, dis_col, b1, w2p)


def _tc_final(q0, q1, hs2p, dis_col, b2):
    n = hs2p.shape[0]
    c_out = b2.shape[0]

    def body(q0_ref, q1_ref, hs_ref, d_ref, b_ref, o_ref):
        t = (q0_ref[...] + q1_ref[...] + hs_ref[...]) * d_ref[...]
        o_ref[...] = t[:, :c_out] + b_ref[...]

    return pl.pallas_call(
        body, out_shape=jax.ShapeDtypeStruct((n, c_out), _F32)
    )(q0, q1, hs2p, dis_col, b2)


def kernel(x, edge_index, edge_attr, W1, b1, W2, b2):
    n, _dim = x.shape
    e = edge_attr.shape[0]
    n_pad = -(-n // (NS * ROWS_PER_TILE)) * (NS * ROWS_PER_TILE)

    align = NC * NS * CHUNK * 8
    e_pad = -(-e // align) * align
    pad = e_pad - e
    src = jnp.pad(edge_index[0], (0, pad)).reshape(-1, CHUNK)
    dst = jnp.pad(edge_index[1], (0, pad)).reshape(-1, CHUNK)
    ew = jnp.pad(edge_attr, (0, pad)).reshape(-1, CHUNK)

    deg_rows = e_pad // (NS * CHUNK)
    agg_rows = e_pad // (NC * NS * CHUNK)

    dis_pad = _make_sc_deg(n_pad, deg_rows)(dst, ew)
    dis_col = dis_pad[:n].reshape(n, 1)

    hs1 = _tc_scale(x, W1, dis_col, n_pad)
    p = _make_sc_agg(n_pad, W1.shape[1], agg_rows)(src, dst, ew, hs1)

    d2p = -(-W2.shape[1] // LANES) * LANES
    w2p = jnp.pad(W2, ((0, 0), (0, d2p - W2.shape[1])))
    hs2p = _tc_mid(p[0, :n], p[1, :n], hs1[:n], dis_col, b1, w2p, n_pad)

    q = _make_sc_agg(n_pad, d2p, agg_rows)(src, dst, ew, hs2p)
    return _tc_final(q[0, :n], q[1, :n], hs2p[:n], dis_col, b2)


# speedup vs baseline: 21.6234x; 1.4940x over previous
import functools

import jax
import jax.numpy as jnp
from jax import lax
from jax.experimental import pallas as pl
from jax.experimental.pallas import tpu as pltpu
from jax.experimental.pallas import tpu_sc as plsc

NC = 2
NS = 16
LANES = 16
CHUNK = 128
ROWS_PER_TILE = 640

_F32 = jnp.float32
_I32 = jnp.int32


def _zero_rows(zb, d):
    z = jnp.zeros((LANES,), _F32)

    def body(i, _):
        for k in range(d // LANES):
            zb[i, pl.ds(k * LANES, LANES)] = z
        return 0

    lax.fori_loop(0, 128, body, 0)


def _make_sc_deg(n_pad, n_chunk_rows):
    seg = n_pad // NS

    @functools.partial(
        pl.kernel,
        mesh=plsc.VectorSubcoreMesh(core_axis_name="c", subcore_axis_name="s"),
        out_type=jax.ShapeDtypeStruct((n_pad,), _F32),
        compiler_params=pltpu.CompilerParams(use_tc_tiling_on_sc=False),
        scratch_types=[
            pltpu.VMEM((n_chunk_rows, CHUNK), _I32),
            pltpu.VMEM((n_chunk_rows, CHUNK), _F32),
            pltpu.VMEM((seg,), _F32),
            pltpu.VMEM_SHARED((n_pad,), _F32),
        ],
    )
    def deg_kernel(dst2d, ew2d, dis_out, dst_all, ew_all, buf, acc):
        c = lax.axis_index("c")
        s = lax.axis_index("s")

        @pl.when(c == 0)
        def _():
            z = jnp.zeros((LANES,), _F32)

            def zb(i, _):
                buf[pl.ds(i * LANES, LANES)] = z
                return 0

            lax.fori_loop(0, seg // LANES, zb, 0)
            pltpu.sync_copy(buf, acc.at[pl.ds(s * seg, seg)])
            plsc.subcore_barrier()

            pltpu.sync_copy(dst2d.at[pl.ds(s * n_chunk_rows, n_chunk_rows)], dst_all)
            pltpu.sync_copy(ew2d.at[pl.ds(s * n_chunk_rows, n_chunk_rows)], ew_all)

            def chunk(g, _):
                pltpu.sync_copy(ew_all.at[g], acc.at[dst_all.at[g]], add=True)
                return 0

            lax.fori_loop(0, n_chunk_rows, chunk, 0)
            plsc.subcore_barrier()

            pltpu.sync_copy(acc.at[pl.ds(s * seg, seg)], buf)

            def newton(i, _):
                sl = pl.ds(i * LANES, LANES)
                d = buf[sl] + 1.0
                pos = d > 0.0
                bits = lax.bitcast_convert_type(d, _I32)
                y = lax.bitcast_convert_type(
                    jnp.int32(0x5F3759DF) - lax.shift_right_arithmetic(bits, 1), _F32
                )
                half = d * 0.5
                for _it in range(4):
                    y = y * (1.5 - half * y * y)
                buf[sl] = jnp.where(pos, y, 0.0)
                return 0

            lax.fori_loop(0, seg // LANES, newton, 0)
            pltpu.sync_copy(buf, dis_out.at[pl.ds(s * seg, seg)])

    return deg_kernel


def _make_sc_agg(n_pad, d, n_chunk_rows):
    seg = n_pad // NS

    @functools.partial(
        pl.kernel,
        mesh=plsc.VectorSubcoreMesh(core_axis_name="c", subcore_axis_name="s"),
        out_type=pltpu.HBM((NC, n_pad, d), _F32),
        compiler_params=pltpu.CompilerParams(use_tc_tiling_on_sc=False),
        scratch_types=[
            pltpu.VMEM((n_chunk_rows, CHUNK), _I32),
            pltpu.VMEM((n_chunk_rows, CHUNK), _I32),
            pltpu.VMEM((n_chunk_rows, CHUNK), _F32),
            pltpu.VMEM((CHUNK, d), _F32),
            pltpu.VMEM((CHUNK, d), _F32),
            pltpu.VMEM((128, d), _F32),
            pltpu.VMEM_SHARED((n_pad, d), _F32),
            pltpu.SemaphoreType.DMA,
            pltpu.SemaphoreType.DMA,
        ],
    )
    def agg_kernel(src2d, dst2d, ew2d, hs, out, src_all, dst_all, ew_all,
                   rows_a, rows_b, zb, acc, sem_a, sem_b):
        c = lax.axis_index("c")
        s = lax.axis_index("s")
        wid = c * NS + s

        _zero_rows(zb, d)
        for i in range(seg // 128):
            pltpu.sync_copy(zb, acc.at[pl.ds(s * seg + i * 128, 128)])
        plsc.subcore_barrier()

        pltpu.sync_copy(src2d.at[pl.ds(wid * n_chunk_rows, n_chunk_rows)], src_all)
        pltpu.sync_copy(dst2d.at[pl.ds(wid * n_chunk_rows, n_chunk_rows)], dst_all)
        pltpu.sync_copy(ew2d.at[pl.ds(wid * n_chunk_rows, n_chunk_rows)], ew_all)

        def gather(g, buf, sem):
            pltpu.async_copy(hs.at[src_all.at[g]], buf, sem)

        def drain(g, buf, sem):
            pltpu.make_async_copy(hs.at[src_all.at[g]], buf, sem).wait()

            def scale(jb, _):
                ewv = ew_all[g, pl.ds(jb * LANES, LANES)]
                for jj in range(LANES):
                    cv = jnp.full((LANES,), ewv[jj], _F32)
                    j = jb * LANES + jj
                    for k in range(d // LANES):
                        sl = pl.ds(k * LANES, LANES)
                        buf[j, sl] = buf[j, sl] * cv
                return 0

            lax.fori_loop(0, CHUNK // LANES, scale, 0)
            pltpu.sync_copy(buf, acc.at[dst_all.at[g]], add=True)

        gather(0, rows_a, sem_a)

        def pair(g2, _):
            g = g2 * 2
            gather(g + 1, rows_b, sem_b)
            drain(g, rows_a, sem_a)
            gather(g + 2, rows_a, sem_a)
            drain(g + 1, rows_b, sem_b)
            return 0

        lax.fori_loop(0, n_chunk_rows // 2 - 1, pair, 0)
        g_last = n_chunk_rows - 2
        gather(g_last + 1, rows_b, sem_b)
        drain(g_last, rows_a, sem_a)
        drain(g_last + 1, rows_b, sem_b)
        plsc.subcore_barrier()

        for i in range(seg // 128):
            r0 = s * seg + i * 128
            pltpu.sync_copy(acc.at[pl.ds(r0, 128)], out.at[c, pl.ds(r0, 128)])

    return agg_kernel


def _tc_scale(x, w, dis_col, n_pad):
    n = x.shape[0]
    h = w.shape[1]

    def body(x_ref, w_ref, d_ref, o_ref):
        hm = jnp.dot(x_ref[...], w_ref[...], preferred_element_type=_F32)
        o_ref[pl.ds(0, n), :] = hm * d_ref[...]
        o_ref[pl.ds(n, n_pad - n), :] = jnp.zeros((n_pad - n, h), _F32)

    return pl.pallas_call(
        body, out_shape=jax.ShapeDtypeStruct((n_pad, h), _F32)
    )(x, w, dis_col)


def _tc_mid(p0, p1, hs1, dis_col, b1, w2p, n_pad):
    n, _h = hs1.shape
    d2 = w2p.shape[1]

    def body(p0_ref, p1_ref, hs_ref, d_ref, b_ref, w_ref, o_ref):
        t = (p0_ref[...] + p1_ref[...] + hs_ref[...]) * d_ref[...] + b_ref[...]
        r = jnp.maximum(t, 0.0)
        o_ref[pl.ds(0, n), :] = (
            jnp.dot(r, w_ref[...], preferred_element_type=_F32) * d_ref[...]
        )
        o_ref[pl.ds(n, n_pad - n), :] = jnp.zeros((n_pad - n, d2), _F32)

    return pl.pallas_call(
        body, out_shape=jax.ShapeDtypeStruct((n_pad, d2), _F32)
    )(p0, p1, hs1, dis_col, b1, w2p)


def _tc_final(q0, q1, hs2p, dis_col, b2):
    n = hs2p.shape[0]
    c_out = b2.shape[0]

    def body(q0_ref, q1_ref, hs_ref, d_ref, b_ref, o_ref):
        t = (q0_ref[...] + q1_ref[...] + hs_ref[...]) * d_ref[...]
        o_ref[...] = t[:, :c_out] + b_ref[...]

    return pl.pallas_call(
        body, out_shape=jax.ShapeDtypeStruct((n, c_out), _F32)
    )(q0, q1, hs2p, dis_col, b2)


def kernel(x, edge_index, edge_attr, W1, b1, W2, b2):
    n, _dim = x.shape
    e = edge_attr.shape[0]
    n_pad = -(-n // (NS * ROWS_PER_TILE)) * (NS * ROWS_PER_TILE)

    align = NC * NS * CHUNK * 8
    e_pad = -(-e // align) * align
    pad = e_pad - e
    src = jnp.pad(edge_index[0], (0, pad)).reshape(-1, CHUNK)
    dst = jnp.pad(edge_index[1], (0, pad)).reshape(-1, CHUNK)
    ew = jnp.pad(edge_attr, (0, pad)).reshape(-1, CHUNK)

    deg_rows = e_pad // (NS * CHUNK)
    agg_rows = e_pad // (NC * NS * CHUNK)

    dis_pad = _make_sc_deg(n_pad, deg_rows)(dst, ew)
    dis_col = dis_pad[:n].reshape(n, 1)

    hs1 = _tc_scale(x, W1, dis_col, n_pad)
    p = _make_sc_agg(n_pad, W1.shape[1], agg_rows)(src, dst, ew, hs1)

    d2p = -(-W2.shape[1] // LANES) * LANES
    w2p = jnp.pad(W2, ((0, 0), (0, d2p - W2.shape[1])))
    hs2p = _tc_mid(p[0, :n], p[1, :n], hs1[:n], dis_col, b1, w2p, n_pad)

    q = _make_sc_agg(n_pad, d2p, agg_rows)(src, dst, ew, hs2p)
    return _tc_final(q[0, :n], q[1, :n], hs2p[:n], dis_col, b2)
